# baseline (speedup 1.0000x reference)
import jax
import jax.numpy as jnp
from jax import lax
from jax.experimental import pallas as pl
from jax.experimental.pallas import tpu as pltpu

B, L, D = 8, 576, 64
K = 1024
N = B * L
DECAY = 0.99
TB = 1152
NBLK = N // TB


def _body(x_ref, x2_ref, cb_ref, counts_ref, ema_ref,
          q_ref, loss_ref, idx_ref, cnew_ref, enew_ref, cbnew_ref,
          enc_sc, c2_sc, cbm2_sc, rows_sc, cbnb_sc, lacc_sc):
    s = pl.program_id(0)

    @pl.when(s < NBLK)
    def _phase_a():
        @pl.when(s == 0)
        def _init():
            cb = cb_ref[...]
            c2_sc[...] = jnp.sum(cb * cb, axis=1, keepdims=True)
            cbm2_sc[...] = -2.0 * cb
            iota_k = lax.broadcasted_iota(jnp.int32, (8, K), 1)
            r = lax.broadcasted_iota(jnp.int32, (8, K), 0)
            rows = jnp.where(
                r == 0, 1.0,
                jnp.where(r == 1, (iota_k // 32).astype(jnp.float32),
                          jnp.where(r == 2, (iota_k % 32).astype(jnp.float32),
                                    0.0)))
            rows_sc[...] = rows.astype(jnp.bfloat16)

        xs = x_ref[pl.ds(s * TB, TB), :]
        xcm2 = lax.dot_general(cbm2_sc[...], xs, (((1,), (1,)), ((), ())),
                               preferred_element_type=jnp.float32)
        x2b = x2_ref[:, pl.ds(s * TB, TB)]
        d = jnp.sqrt(jnp.maximum((x2b + xcm2) + c2_sc[...], 0.0))
        dmin = jnp.min(d, axis=0, keepdims=True)
        maskb = (d == dmin).astype(jnp.bfloat16)
        enc_sc[:, pl.ds(s * TB, TB)] = maskb

        stat = lax.dot_general(rows_sc[...], maskb, (((1,), (0,)), ((), ())),
                               preferred_element_type=jnp.float32)
        idxf = 32.0 * stat[1:2, :] + stat[2:3, :]
        idx_ref[:, pl.ds(s * TB, TB)] = idxf.astype(jnp.int32)

        @pl.when(jnp.max(stat[0:1, :]) > 1.5)
        def _tie_fix():
            iota_sub = lax.broadcasted_iota(jnp.int32, (K, TB), 0)
            idx_ex = jnp.min(jnp.where(d == dmin, iota_sub, K), axis=0,
                             keepdims=True)
            enc_sc[:, pl.ds(s * TB, TB)] = (iota_sub == idx_ex).astype(
                jnp.bfloat16)
            idx_ref[:, pl.ds(s * TB, TB)] = idx_ex

        @pl.when(s == NBLK - 1)
        def _update():
            xb = x_ref[...].astype(jnp.bfloat16)
            aug = jnp.concatenate(
                [xb, jnp.ones((N, 1), jnp.bfloat16),
                 jnp.zeros((N, 128 - D - 1), jnp.bfloat16)], axis=1)
            dw_aug = lax.dot_general(enc_sc[...], aug,
                                     (((1,), (0,)), ((), ())),
                                     preferred_element_type=jnp.float32)
            dw = dw_aug[:, :D]
            hist = dw_aug[:, D:D + 1]
            cnt_col = jnp.transpose(counts_ref[...])
            cnew = DECAY * cnt_col + (1.0 - DECAY) * hist
            enew = DECAY * ema_ref[...] + (1.0 - DECAY) * dw
            cbnew = enew / cnew
            cnew_ref[...] = jnp.transpose(cnew)
            enew_ref[...] = enew
            cbnew_ref[...] = cbnew
            cbnb_sc[...] = cbnew.astype(jnp.bfloat16)

    @pl.when(s == NBLK)
    def _phase_b():
        q = lax.dot_general(enc_sc[...], cbnb_sc[...],
                            (((0,), (0,)), ((), ())),
                            preferred_element_type=jnp.float32)
        q_ref[...] = q
        xs = x_ref[...]
        loss_ref[...] = (0.5 * jnp.sum(jnp.square(xs - q))
                         / (N * D)).reshape(1, 1)


def _fused(x2d, x2row, codebook, counts_row, ema_weight):
    return pl.pallas_call(
        _body,
        grid=(NBLK + 1,),
        in_specs=[
            pl.BlockSpec((N, D), lambda s: (0, 0)),
            pl.BlockSpec((1, N), lambda s: (0, 0)),
            pl.BlockSpec((K, D), lambda s: (0, 0)),
            pl.BlockSpec((1, K), lambda s: (0, 0)),
            pl.BlockSpec((K, D), lambda s: (0, 0)),
        ],
        out_specs=[
            pl.BlockSpec((N, D), lambda s: (0, 0)),
            pl.BlockSpec((1, 1), lambda s: (0, 0)),
            pl.BlockSpec((1, N), lambda s: (0, 0)),
            pl.BlockSpec((1, K), lambda s: (0, 0)),
            pl.BlockSpec((K, D), lambda s: (0, 0)),
            pl.BlockSpec((K, D), lambda s: (0, 0)),
        ],
        out_shape=[
            jax.ShapeDtypeStruct((N, D), jnp.float32),
            jax.ShapeDtypeStruct((1, 1), jnp.float32),
            jax.ShapeDtypeStruct((1, N), jnp.int32),
            jax.ShapeDtypeStruct((1, K), jnp.float32),
            jax.ShapeDtypeStruct((K, D), jnp.float32),
            jax.ShapeDtypeStruct((K, D), jnp.float32),
        ],
        scratch_shapes=[
            pltpu.VMEM((K, N), jnp.bfloat16),
            pltpu.VMEM((K, 1), jnp.float32),
            pltpu.VMEM((K, D), jnp.float32),
            pltpu.VMEM((8, K), jnp.bfloat16),
            pltpu.VMEM((K, D), jnp.bfloat16),
            pltpu.VMEM((1, D), jnp.float32),
        ],
    )(x2d, x2row, codebook, counts_row, ema_weight)


def kernel(x, codebook, ema_weight, counts):
    x2d = x.reshape(N, D)
    x2row = jnp.sum(x2d * x2d, axis=1).reshape(1, N)
    q2d, loss, idx, cnew, enew, cbnew = _fused(
        x2d, x2row, codebook, counts.reshape(1, K), ema_weight)
    return (q2d.reshape(B, L, D), loss[0, 0], idx.reshape(B, L),
            cnew.reshape(K), enew, cbnew)
